# baseline (speedup 1.0000x reference)
import jax
import jax.numpy as jnp
from jax import lax
from jax.experimental import pallas as pl
from jax.experimental.pallas import tpu as pltpu
from jax.experimental.pallas import tpu_sc as plsc

_VOCAB = 100000
_D = 128
_B = 32
_M = 200
_TOK = 4
_NT = 50
_NC = 2
_NS = 16
_NW = _NC * _NS
_CHUNK = 100
_NCHUNK = _M // _CHUNK
_ROWS_SC = _NS * _M
_TV = 7168
_NV = (_VOCAB + _TV - 1) // _TV
_NVH = _NV // 2


def _sc_table(table, idx_v, oidx_v, rows_v, acc_v, ab, base,
              gsem, ssem, asem):
    def gat(k, b):
        return pltpu.async_copy(table.at[idx_v.at[k]], rows_v.at[b], gsem)

    def add(b, p):
        return pltpu.async_copy(rows_v.at[b], acc_v.at[oidx_v.at[ab, p]], asem,
                                add=True)

    g = {k: gat(k, k) for k in range(4)}
    g[0].wait()
    s0 = pltpu.async_copy(rows_v.at[0], acc_v.at[pl.ds(base, _CHUNK)], ssem)
    g[1].wait()
    s1 = pltpu.async_copy(rows_v.at[1], acc_v.at[pl.ds(base + _CHUNK, _CHUNK)],
                          ssem)
    s0.wait()
    s1.wait()
    g[4] = gat(4, 0)
    g[5] = gat(5, 1)
    g[2].wait()
    a2 = add(2, 0)
    g[3].wait()
    a3 = add(3, 1)
    a2.wait()
    a3.wait()
    g[6] = gat(6, 2)
    g[7] = gat(7, 3)
    g[4].wait()
    a4 = add(0, 0)
    g[5].wait()
    a5 = add(1, 1)
    g[6].wait()
    a6 = add(2, 0)
    g[7].wait()
    a7 = add(3, 1)
    for a in (a4, a5, a6, a7):
        a.wait()


def _sc1_body(idx_hbm, oidx_hbm, dec_hbm, c0, c1, m_out, x_out,
              idx_v, oidx_v, dec_v, rows_v, acc_v, gsem, ssem, asem, outsem):
    c = lax.axis_index("c")
    s = lax.axis_index("s")
    w = c * _NS + s
    pltpu.sync_copy(idx_hbm.at[w], idx_v)
    pltpu.sync_copy(oidx_hbm.at[s], oidx_v)

    _sc_table(c0, idx_v, oidx_v, rows_v, acc_v, 0, s * _M, gsem, ssem, asem)
    out0 = pltpu.async_copy(acc_v.at[pl.ds(s * _M, _M)],
                            m_out.at[0, pl.ds(w * _M, _M)], outsem)
    _sc_table(c1, idx_v, oidx_v, rows_v, acc_v, 1, _ROWS_SC + s * _M,
              gsem, ssem, asem)
    out1 = pltpu.async_copy(acc_v.at[pl.ds(_ROWS_SC + s * _M, _M)],
                            m_out.at[1, pl.ds(w * _M, _M)], outsem)

    @pl.when(jnp.logical_and(c == 0, s == 0))
    def _():
        pltpu.sync_copy(dec_hbm, dec_v)
        pltpu.sync_copy(c0.at[dec_v.at[0]], rows_v.at[0, pl.ds(0, _B)])
        pltpu.sync_copy(rows_v.at[0, pl.ds(0, _B)], x_out)

    out0.wait()
    out1.wait()


def _sc2_body(idx_hbm, oidx_hbm, c2, m_out,
              idx_v, oidx_v, rows_v, acc_v, gsem, ssem, asem, outsem):
    c = lax.axis_index("c")
    s = lax.axis_index("s")
    w = c * _NS + s
    pltpu.sync_copy(idx_hbm.at[w], idx_v)
    pltpu.sync_copy(oidx_hbm.at[s], oidx_v)
    _sc_table(c2, idx_v, oidx_v, rows_v, acc_v, 0, s * _M, gsem, ssem, asem)
    pltpu.sync_copy(acc_v.at[pl.ds(s * _M, _M)], m_out.at[pl.ds(w * _M, _M)])


_SC_MESH = dict(core_axis_name="c", subcore_axis_name="s")


def _sc_gather1(idx, oidx, dec, C0, C1):
    fn = pl.kernel(
        _sc1_body,
        mesh=plsc.VectorSubcoreMesh(**_SC_MESH),
        out_type=[jax.ShapeDtypeStruct((2, _B * _M, _D), jnp.float32),
                  jax.ShapeDtypeStruct((_B, _D), jnp.float32)],
        scratch_types=[pltpu.VMEM((2 * _TOK, _CHUNK), jnp.int32),
                       pltpu.VMEM((2, _NCHUNK, _CHUNK), jnp.int32),
                       pltpu.VMEM((1, _B), jnp.int32),
                       pltpu.VMEM((4, _CHUNK, _D), jnp.float32),
                       pltpu.VMEM_SHARED((2 * _ROWS_SC, _D), jnp.float32),
                       pltpu.SemaphoreType.DMA,
                       pltpu.SemaphoreType.DMA,
                       pltpu.SemaphoreType.DMA,
                       pltpu.SemaphoreType.DMA],
    )
    return fn(idx, oidx, dec, C0, C1)


def _sc_gather2(idx, oidx, C2):
    fn = pl.kernel(
        _sc2_body,
        mesh=plsc.VectorSubcoreMesh(**_SC_MESH),
        out_type=jax.ShapeDtypeStruct((_B * _M, _D), jnp.float32),
        scratch_types=[pltpu.VMEM((2 * _TOK, _CHUNK), jnp.int32),
                       pltpu.VMEM((2, _NCHUNK, _CHUNK), jnp.int32),
                       pltpu.VMEM((4, _CHUNK, _D), jnp.float32),
                       pltpu.VMEM_SHARED((_ROWS_SC, _D), jnp.float32),
                       pltpu.SemaphoreType.DMA,
                       pltpu.SemaphoreType.DMA,
                       pltpu.SemaphoreType.DMA,
                       pltpu.SemaphoreType.DMA],
    )
    return fn(idx, oidx, C2)


def _dense1_body(x_ref, h0_ref, roots_ref, bias_ref, m_ref,
                 wq_ref, wk_ref, wv_ref, wih_ref, whh_ref, bih_ref, bhh_ref,
                 cur_ref, uo_ref):
    h0 = h0_ref[...]
    q = lax.dot(h0, wq_ref[...], preferred_element_type=jnp.float32)
    roots = roots_ref[...]
    roots2 = roots.reshape(_B * _NT, _D)
    rk = lax.dot(roots2, wk_ref[...], preferred_element_type=jnp.float32)
    rv = lax.dot(roots2, wv_ref[...], preferred_element_type=jnp.float32)
    rk = rk.reshape(_B, _NT, _D)
    rv = rv.reshape(_B, _NT, _D)
    rk_b = rk.astype(jnp.bfloat16).astype(jnp.float32)
    q_b = q.astype(jnp.bfloat16).astype(jnp.float32)
    al = jnp.sum(rk_b * q_b[:, None, :], axis=2) + bias_ref[...]
    aw = jax.nn.softmax(al, axis=1)
    kb = jnp.sum(aw[:, :, None] * rv, axis=1)

    x = x_ref[...]
    gi = lax.dot(x, wih_ref[...], preferred_element_type=jnp.float32) + bih_ref[...]
    gh = lax.dot(h0, whh_ref[...], preferred_element_type=jnp.float32) + bhh_ref[...]
    r = jax.nn.sigmoid(gi[:, 0:_D] + gh[:, 0:_D])
    z = jax.nn.sigmoid(gi[:, _D:2 * _D] + gh[:, _D:2 * _D])
    n = jnp.tanh(gi[:, 2 * _D:3 * _D] + r * gh[:, 2 * _D:3 * _D])
    hidden = (1.0 - z) * n + z * h0
    u = hidden + kb
    cur_ref[...] = u

    logits = jnp.sum(m_ref[0] * u[:, None, :], axis=2)
    prob = jax.nn.softmax(logits, axis=1)
    o_k = jnp.sum(m_ref[1] * prob[:, :, None], axis=1)
    uo_ref[:, 0:_D] = u
    uo_ref[:, _D:2 * _D] = o_k


def _dense1(x, h0, roots, bias, m01, Wq, Wk, Wv, W_ih, W_hh, b_ih2, b_hh2):
    return pl.pallas_call(
        _dense1_body,
        out_shape=[jax.ShapeDtypeStruct((_B, _D), jnp.float32),
                   jax.ShapeDtypeStruct((_B, 2 * _D), jnp.float32)],
    )(x, h0, roots, bias, m01, Wq, Wk, Wv, W_ih, W_hh, b_ih2, b_hh2)


def _dense2_body(uo_ref, m01_ref, m2_ref, pptr_ref):
    u1 = uo_ref[:, 0:_D] + uo_ref[:, _D:2 * _D]
    m1 = m01_ref[1]
    logits1 = jnp.sum(m1 * u1[:, None, :], axis=2)
    prob1 = jax.nn.softmax(logits1, axis=1)
    m2 = m2_ref[...]
    o_k1 = jnp.sum(m2 * prob1[:, :, None], axis=1)
    u2 = u1 + o_k1
    pptr_ref[...] = jnp.sum(m2 * u2[:, None, :], axis=2)


def _dense2(uo, m01, m2):
    return pl.pallas_call(
        _dense2_body,
        out_shape=jax.ShapeDtypeStruct((_B, _M), jnp.float32),
    )(uo, m01, m2.reshape(_B, _M, _D))


def _vocab_body(uo_ref, w_ref, b_ref, e_ref, ms_ref, logit_ref, m_ref, s_ref):
    k = pl.program_id(0)
    i = pl.program_id(1)

    @pl.when(i == 0)
    def _():
        m_ref[...] = jnp.full((_B, 128), -3e38, jnp.float32)
        s_ref[...] = jnp.zeros((_B, 128), jnp.float32)

    @pl.when(i < _NVH)
    def _():
        logits = lax.dot_general(uo_ref[...], w_ref[...],
                                 (((1,), (1,)), ((), ())),
                                 preferred_element_type=jnp.float32)
        logits = logits + b_ref[...]
        col = ((k * _NVH + i) * _TV
               + lax.broadcasted_iota(jnp.int32, (_B, _TV), 1))
        logits = jnp.where(col < _VOCAB, logits, -1e30)
        logit_ref[i] = logits
        t_max = jnp.max(logits, axis=1, keepdims=True)
        m_old = m_ref[...]
        m_new = jnp.maximum(m_old, t_max)
        ssum = jnp.sum(jnp.exp(logits - m_new[:, :1]), axis=1, keepdims=True)
        s_ref[...] = s_ref[...] * jnp.exp(m_old - m_new) + ssum
        m_ref[...] = m_new

        @pl.when(i == _NVH - 1)
        def _():
            ms_ref[0, 0] = m_ref[...]
            ms_ref[0, 1] = s_ref[...]

    @pl.when(i >= _NVH)
    def _():
        lg = logit_ref[i - _NVH]
        e_ref[...] = jnp.exp(lg - m_ref[:, :1])


def _vocab(uo, W1t, b12):
    return pl.pallas_call(
        _vocab_body,
        grid=(2, 2 * _NVH),
        in_specs=[
            pl.BlockSpec((_B, 2 * _D), lambda k, i: (0, 0)),
            pl.BlockSpec((_TV, 2 * _D),
                         lambda k, i: (k * _NVH + lax.min(i, _NVH - 1), 0)),
            pl.BlockSpec((1, _TV),
                         lambda k, i: (0, k * _NVH + lax.min(i, _NVH - 1))),
        ],
        out_specs=[
            pl.BlockSpec((_B, _TV),
                         lambda k, i: (0, k * _NVH + lax.max(i - _NVH, 0))),
            pl.BlockSpec((1, 2, _B, 128), lambda k, i: (k, 0, 0, 0)),
        ],
        out_shape=[jax.ShapeDtypeStruct((_B, _VOCAB), jnp.float32),
                   jax.ShapeDtypeStruct((2, 2, _B, 128), jnp.float32)],
        scratch_shapes=[pltpu.VMEM((_NVH, _B, _TV), jnp.float32),
                        pltpu.VMEM((_B, 128), jnp.float32),
                        pltpu.VMEM((_B, 128), jnp.float32)],
        compiler_params=pltpu.CompilerParams(
            dimension_semantics=("parallel", "arbitrary")),
    )(uo, W1t, b12)


def _rescale_body(e_ref, ms_ref, p_ref):
    k = pl.program_id(0)
    m0 = ms_ref[0, 0, :, 0:1]
    s0 = ms_ref[0, 1, :, 0:1]
    m1 = ms_ref[1, 0, :, 0:1]
    s1 = ms_ref[1, 1, :, 0:1]
    mg = jnp.maximum(m0, m1)
    sg = s0 * jnp.exp(m0 - mg) + s1 * jnp.exp(m1 - mg)
    mk = jnp.where(k == 0, m0, m1)
    p_ref[...] = e_ref[...] * (jnp.exp(mk - mg) / sg)


def _rescale(e, ms):
    return pl.pallas_call(
        _rescale_body,
        grid=(2, _NVH),
        in_specs=[
            pl.BlockSpec((_B, _TV), lambda k, i: (0, k * _NVH + i)),
            pl.BlockSpec((2, 2, _B, 128), lambda k, i: (0, 0, 0, 0)),
        ],
        out_specs=pl.BlockSpec((_B, _TV), lambda k, i: (0, k * _NVH + i)),
        out_shape=jax.ShapeDtypeStruct((_B, _VOCAB), jnp.float32),
        compiler_params=pltpu.CompilerParams(
            dimension_semantics=("parallel", "arbitrary")),
    )(e, ms)


def kernel(decoder_input, story, hidden_states, roots_embed, attention_bias,
           global_index, C0, C1, C2, C3, Wq, Wk, Wv, W1, b1,
           W_ih, W_hh, b_ih, b_hh):
    story = story.astype(jnp.int32)
    dec = decoder_input.astype(jnp.int32).reshape(1, _B)
    idx = story.transpose(0, 2, 1).reshape(_NW, _TOK * _NCHUNK, _CHUNK)
    oidx = (jnp.arange(2, dtype=jnp.int32)[None, :, None, None] * _ROWS_SC
            + jnp.arange(_NS, dtype=jnp.int32)[:, None, None, None] * _M
            + jnp.arange(_NCHUNK, dtype=jnp.int32)[None, None, :, None] * _CHUNK
            + jnp.arange(_CHUNK, dtype=jnp.int32)[None, None, None, :])
    m01_flat, x = _sc_gather1(idx, oidx, dec, C0, C1)
    m2_flat = _sc_gather2(idx, oidx, C2)
    m01 = m01_flat.reshape(2, _B, _M, _D)
    h0 = hidden_states[0]
    bias = attention_bias[:, :, 0]
    cur, uo = _dense1(x, h0, roots_embed, bias, m01, Wq, Wk, Wv,
                      W_ih, W_hh, b_ih.reshape(1, -1), b_hh.reshape(1, -1))
    e, ms = _vocab(uo, W1.T, b1.reshape(1, -1))
    pvocab = _rescale(e, ms)
    pptr = _dense2(uo, m01, m2_flat)
    return (pptr, pvocab, cur[None])


# speedup vs baseline: 1.2148x; 1.2148x over previous
import jax
import jax.numpy as jnp
from jax import lax
from jax.experimental import pallas as pl
from jax.experimental.pallas import tpu as pltpu
from jax.experimental.pallas import tpu_sc as plsc

_VOCAB = 100000
_D = 128
_B = 32
_M = 200
_TOK = 4
_NT = 50
_NC = 2
_NS = 16
_NW = _NC * _NS
_CHUNK = 100
_NCHUNK = _M // _CHUNK
_ROWS_SC = _NS * _M
_TV = 8192
_NV = (_VOCAB + _TV - 1) // _TV


def _sc_table(table, idx_v, oidx_v, rows_v, acc_v, ab, base,
              gsem, ssem, asem):
    def gat(k, b):
        return pltpu.async_copy(table.at[idx_v.at[k]], rows_v.at[b], gsem)

    def add(b, p):
        return pltpu.async_copy(rows_v.at[b], acc_v.at[oidx_v.at[ab, p]], asem,
                                add=True)

    g = {k: gat(k, k) for k in range(4)}
    g[0].wait()
    s0 = pltpu.async_copy(rows_v.at[0], acc_v.at[pl.ds(base, _CHUNK)], ssem)
    g[1].wait()
    s1 = pltpu.async_copy(rows_v.at[1], acc_v.at[pl.ds(base + _CHUNK, _CHUNK)],
                          ssem)
    s0.wait()
    s1.wait()
    g[4] = gat(4, 0)
    g[5] = gat(5, 1)
    g[2].wait()
    a2 = add(2, 0)
    g[3].wait()
    a3 = add(3, 1)
    a2.wait()
    a3.wait()
    g[6] = gat(6, 2)
    g[7] = gat(7, 3)
    g[4].wait()
    a4 = add(0, 0)
    g[5].wait()
    a5 = add(1, 1)
    g[6].wait()
    a6 = add(2, 0)
    g[7].wait()
    a7 = add(3, 1)
    for a in (a4, a5, a6, a7):
        a.wait()


def _sc1_body(idx_hbm, oidx_hbm, dec_hbm, c0, c1, m_out, x_out,
              idx_v, oidx_v, dec_v, rows_v, acc_v, gsem, ssem, asem, outsem):
    c = lax.axis_index("c")
    s = lax.axis_index("s")
    w = c * _NS + s
    pltpu.sync_copy(idx_hbm.at[w], idx_v)
    pltpu.sync_copy(oidx_hbm.at[s], oidx_v)

    _sc_table(c0, idx_v, oidx_v, rows_v, acc_v, 0, s * _M, gsem, ssem, asem)
    out0 = pltpu.async_copy(acc_v.at[pl.ds(s * _M, _M)],
                            m_out.at[0, pl.ds(w * _M, _M)], outsem)
    _sc_table(c1, idx_v, oidx_v, rows_v, acc_v, 1, _ROWS_SC + s * _M,
              gsem, ssem, asem)
    out1 = pltpu.async_copy(acc_v.at[pl.ds(_ROWS_SC + s * _M, _M)],
                            m_out.at[1, pl.ds(w * _M, _M)], outsem)

    @pl.when(jnp.logical_and(c == 0, s == 0))
    def _():
        pltpu.sync_copy(dec_hbm, dec_v)
        pltpu.sync_copy(c0.at[dec_v.at[0]], rows_v.at[0, pl.ds(0, _B)])
        pltpu.sync_copy(rows_v.at[0, pl.ds(0, _B)], x_out)

    out0.wait()
    out1.wait()


def _sc2_body(idx_hbm, oidx_hbm, c2, m_out,
              idx_v, oidx_v, rows_v, acc_v, gsem, ssem, asem, outsem):
    c = lax.axis_index("c")
    s = lax.axis_index("s")
    w = c * _NS + s
    pltpu.sync_copy(idx_hbm.at[w], idx_v)
    pltpu.sync_copy(oidx_hbm.at[s], oidx_v)
    _sc_table(c2, idx_v, oidx_v, rows_v, acc_v, 0, s * _M, gsem, ssem, asem)
    pltpu.sync_copy(acc_v.at[pl.ds(s * _M, _M)], m_out.at[pl.ds(w * _M, _M)])


_SC_MESH = dict(core_axis_name="c", subcore_axis_name="s")


def _sc_gather1(idx, oidx, dec, C0, C1):
    fn = pl.kernel(
        _sc1_body,
        mesh=plsc.VectorSubcoreMesh(**_SC_MESH),
        out_type=[jax.ShapeDtypeStruct((2, _B * _M, _D), jnp.float32),
                  jax.ShapeDtypeStruct((_B, _D), jnp.float32)],
        scratch_types=[pltpu.VMEM((2 * _TOK, _CHUNK), jnp.int32),
                       pltpu.VMEM((2, _NCHUNK, _CHUNK), jnp.int32),
                       pltpu.VMEM((1, _B), jnp.int32),
                       pltpu.VMEM((4, _CHUNK, _D), jnp.float32),
                       pltpu.VMEM_SHARED((2 * _ROWS_SC, _D), jnp.float32),
                       pltpu.SemaphoreType.DMA,
                       pltpu.SemaphoreType.DMA,
                       pltpu.SemaphoreType.DMA,
                       pltpu.SemaphoreType.DMA],
    )
    return fn(idx, oidx, dec, C0, C1)


def _sc_gather2(idx, oidx, C2):
    fn = pl.kernel(
        _sc2_body,
        mesh=plsc.VectorSubcoreMesh(**_SC_MESH),
        out_type=jax.ShapeDtypeStruct((_B * _M, _D), jnp.float32),
        scratch_types=[pltpu.VMEM((2 * _TOK, _CHUNK), jnp.int32),
                       pltpu.VMEM((2, _NCHUNK, _CHUNK), jnp.int32),
                       pltpu.VMEM((4, _CHUNK, _D), jnp.float32),
                       pltpu.VMEM_SHARED((_ROWS_SC, _D), jnp.float32),
                       pltpu.SemaphoreType.DMA,
                       pltpu.SemaphoreType.DMA,
                       pltpu.SemaphoreType.DMA,
                       pltpu.SemaphoreType.DMA],
    )
    return fn(idx, oidx, C2)


def _dense1_body(x_ref, h0_ref, roots_ref, bias_ref, m_ref,
                 wq_ref, wk_ref, wv_ref, wih_ref, whh_ref, bih_ref, bhh_ref,
                 cur_ref, uo_ref):
    h0 = h0_ref[...]
    q = lax.dot(h0, wq_ref[...], preferred_element_type=jnp.float32)
    roots = roots_ref[...]
    roots2 = roots.reshape(_B * _NT, _D)
    rk = lax.dot(roots2, wk_ref[...], preferred_element_type=jnp.float32)
    rv = lax.dot(roots2, wv_ref[...], preferred_element_type=jnp.float32)
    rk = rk.reshape(_B, _NT, _D)
    rv = rv.reshape(_B, _NT, _D)
    rk_b = rk.astype(jnp.bfloat16).astype(jnp.float32)
    q_b = q.astype(jnp.bfloat16).astype(jnp.float32)
    al = jnp.sum(rk_b * q_b[:, None, :], axis=2) + bias_ref[...]
    aw = jax.nn.softmax(al, axis=1)
    kb = jnp.sum(aw[:, :, None] * rv, axis=1)

    x = x_ref[...]
    gi = lax.dot(x, wih_ref[...], preferred_element_type=jnp.float32) + bih_ref[...]
    gh = lax.dot(h0, whh_ref[...], preferred_element_type=jnp.float32) + bhh_ref[...]
    r = jax.nn.sigmoid(gi[:, 0:_D] + gh[:, 0:_D])
    z = jax.nn.sigmoid(gi[:, _D:2 * _D] + gh[:, _D:2 * _D])
    n = jnp.tanh(gi[:, 2 * _D:3 * _D] + r * gh[:, 2 * _D:3 * _D])
    hidden = (1.0 - z) * n + z * h0
    u = hidden + kb
    cur_ref[...] = u

    logits = jnp.sum(m_ref[0] * u[:, None, :], axis=2)
    prob = jax.nn.softmax(logits, axis=1)
    o_k = jnp.sum(m_ref[1] * prob[:, :, None], axis=1)
    uo_ref[:, 0:_D] = u
    uo_ref[:, _D:2 * _D] = o_k


def _dense1(x, h0, roots, bias, m01, Wq, Wk, Wv, W_ih, W_hh, b_ih2, b_hh2):
    return pl.pallas_call(
        _dense1_body,
        out_shape=[jax.ShapeDtypeStruct((_B, _D), jnp.float32),
                   jax.ShapeDtypeStruct((_B, 2 * _D), jnp.float32)],
    )(x, h0, roots, bias, m01, Wq, Wk, Wv, W_ih, W_hh, b_ih2, b_hh2)


def _dense2_body(uo_ref, m01_ref, m2_ref, pptr_ref):
    u1 = uo_ref[:, 0:_D] + uo_ref[:, _D:2 * _D]
    m1 = m01_ref[1]
    logits1 = jnp.sum(m1 * u1[:, None, :], axis=2)
    prob1 = jax.nn.softmax(logits1, axis=1)
    m2 = m2_ref[...]
    o_k1 = jnp.sum(m2 * prob1[:, :, None], axis=1)
    u2 = u1 + o_k1
    pptr_ref[...] = jnp.sum(m2 * u2[:, None, :], axis=2)


def _dense2(uo, m01, m2):
    return pl.pallas_call(
        _dense2_body,
        out_shape=jax.ShapeDtypeStruct((_B, _M), jnp.float32),
    )(uo, m01, m2.reshape(_B, _M, _D))


def _vocab_body(uo_ref, w_ref, b_ref, out_ref, e_ref, mt_ref, m_ref, s_ref):
    i = pl.program_id(0)

    @pl.when(i == 0)
    def _():
        m_ref[...] = jnp.full((_B, 128), -3e38, jnp.float32)
        s_ref[...] = jnp.zeros((_B, 128), jnp.float32)

    @pl.when(i < _NV)
    def _():
        logits = lax.dot_general(uo_ref[...], w_ref[...],
                                 (((1,), (1,)), ((), ())),
                                 preferred_element_type=jnp.float32)
        logits = logits + b_ref[...]
        col = i * _TV + lax.broadcasted_iota(jnp.int32, (_B, _TV), 1)
        logits = jnp.where(col < _VOCAB, logits, -1e30)
        t_max = jnp.max(logits, axis=1, keepdims=True)
        m_old = m_ref[...]
        m_new = jnp.maximum(m_old, t_max)
        e = jnp.exp(logits - m_new[:, :1])
        e_ref[i] = e
        mt_ref[i] = m_new
        ssum = jnp.sum(e, axis=1, keepdims=True)
        s_ref[...] = s_ref[...] * jnp.exp(m_old - m_new) + ssum
        m_ref[...] = m_new

    @pl.when(i >= _NV)
    def _():
        j = i - _NV
        scale = jnp.exp(mt_ref[j, :, 0:1] - m_ref[:, 0:1]) / s_ref[:, 0:1]
        out_ref[...] = e_ref[j] * scale


def _vocab(uo, W1t, b12):
    return pl.pallas_call(
        _vocab_body,
        grid=(2 * _NV,),
        in_specs=[
            pl.BlockSpec((_B, 2 * _D), lambda i: (0, 0)),
            pl.BlockSpec((_TV, 2 * _D), lambda i: (lax.min(i, _NV - 1), 0)),
            pl.BlockSpec((1, _TV), lambda i: (0, lax.min(i, _NV - 1))),
        ],
        out_specs=pl.BlockSpec((_B, _TV), lambda i: (0, lax.max(i - _NV, 0))),
        out_shape=jax.ShapeDtypeStruct((_B, _VOCAB), jnp.float32),
        scratch_shapes=[pltpu.VMEM((_NV, _B, _TV), jnp.float32),
                        pltpu.VMEM((_NV, _B, 128), jnp.float32),
                        pltpu.VMEM((_B, 128), jnp.float32),
                        pltpu.VMEM((_B, 128), jnp.float32)],
    )(uo, W1t, b12)


def kernel(decoder_input, story, hidden_states, roots_embed, attention_bias,
           global_index, C0, C1, C2, C3, Wq, Wk, Wv, W1, b1,
           W_ih, W_hh, b_ih, b_hh):
    story = story.astype(jnp.int32)
    dec = decoder_input.astype(jnp.int32).reshape(1, _B)
    idx = story.transpose(0, 2, 1).reshape(_NW, _TOK * _NCHUNK, _CHUNK)
    oidx = (jnp.arange(2, dtype=jnp.int32)[None, :, None, None] * _ROWS_SC
            + jnp.arange(_NS, dtype=jnp.int32)[:, None, None, None] * _M
            + jnp.arange(_NCHUNK, dtype=jnp.int32)[None, None, :, None] * _CHUNK
            + jnp.arange(_CHUNK, dtype=jnp.int32)[None, None, None, :])
    m01_flat, x = _sc_gather1(idx, oidx, dec, C0, C1)
    m2_flat = _sc_gather2(idx, oidx, C2)
    m01 = m01_flat.reshape(2, _B, _M, _D)
    h0 = hidden_states[0]
    bias = attention_bias[:, :, 0]
    cur, uo = _dense1(x, h0, roots_embed, bias, m01, Wq, Wk, Wv,
                      W_ih, W_hh, b_ih.reshape(1, -1), b_hh.reshape(1, -1))
    pvocab = _vocab(uo, W1.T, b1.reshape(1, -1))
    uo2, _pv = lax.optimization_barrier((uo, pvocab))
    pptr = _dense2(uo2, m01, m2_flat)
    pvocab = _pv
    return (pptr, pvocab, cur[None])


# speedup vs baseline: 1.2345x; 1.0162x over previous
import jax
import jax.numpy as jnp
from jax import lax
from jax.experimental import pallas as pl
from jax.experimental.pallas import tpu as pltpu
from jax.experimental.pallas import tpu_sc as plsc

_VOCAB = 100000
_D = 128
_B = 32
_M = 200
_TOK = 4
_NT = 50
_NC = 2
_NS = 16
_NW = _NC * _NS
_CHUNK = 100
_NCHUNK = _M // _CHUNK
_ROWS_SC = _NS * _M
_TV = 12288
_NV = (_VOCAB + _TV - 1) // _TV


def _sc_table(table, idx_v, oidx_v, rows_v, acc_v, ab, base,
              gsem, ssem, asem):
    def gat(k, b):
        return pltpu.async_copy(table.at[idx_v.at[k]], rows_v.at[b], gsem)

    def add(b, p):
        return pltpu.async_copy(rows_v.at[b], acc_v.at[oidx_v.at[ab, p]], asem,
                                add=True)

    g = {k: gat(k, k) for k in range(4)}
    g[0].wait()
    s0 = pltpu.async_copy(rows_v.at[0], acc_v.at[pl.ds(base, _CHUNK)], ssem)
    g[1].wait()
    s1 = pltpu.async_copy(rows_v.at[1], acc_v.at[pl.ds(base + _CHUNK, _CHUNK)],
                          ssem)
    s0.wait()
    s1.wait()
    g[4] = gat(4, 0)
    g[5] = gat(5, 1)
    g[2].wait()
    a2 = add(2, 0)
    g[3].wait()
    a3 = add(3, 1)
    a2.wait()
    a3.wait()
    g[6] = gat(6, 2)
    g[7] = gat(7, 3)
    g[4].wait()
    a4 = add(0, 0)
    g[5].wait()
    a5 = add(1, 1)
    g[6].wait()
    a6 = add(2, 0)
    g[7].wait()
    a7 = add(3, 1)
    for a in (a4, a5, a6, a7):
        a.wait()


def _sc1_body(idx_hbm, oidx_hbm, dec_hbm, c0, c1, m_out, x_out,
              idx_v, oidx_v, dec_v, rows_v, acc_v, gsem, ssem, asem, outsem):
    c = lax.axis_index("c")
    s = lax.axis_index("s")
    w = c * _NS + s
    pltpu.sync_copy(idx_hbm.at[w], idx_v)
    pltpu.sync_copy(oidx_hbm.at[s], oidx_v)

    _sc_table(c0, idx_v, oidx_v, rows_v, acc_v, 0, s * _M, gsem, ssem, asem)
    out0 = pltpu.async_copy(acc_v.at[pl.ds(s * _M, _M)],
                            m_out.at[0, pl.ds(w * _M, _M)], outsem)
    _sc_table(c1, idx_v, oidx_v, rows_v, acc_v, 1, _ROWS_SC + s * _M,
              gsem, ssem, asem)
    out1 = pltpu.async_copy(acc_v.at[pl.ds(_ROWS_SC + s * _M, _M)],
                            m_out.at[1, pl.ds(w * _M, _M)], outsem)

    @pl.when(jnp.logical_and(c == 0, s == 0))
    def _():
        pltpu.sync_copy(dec_hbm, dec_v)
        pltpu.sync_copy(c0.at[dec_v.at[0]], rows_v.at[0, pl.ds(0, _B)])
        pltpu.sync_copy(rows_v.at[0, pl.ds(0, _B)], x_out)

    out0.wait()
    out1.wait()


def _sc2_body(idx_hbm, oidx_hbm, c2, m_out,
              idx_v, oidx_v, rows_v, acc_v, gsem, ssem, asem, outsem):
    c = lax.axis_index("c")
    s = lax.axis_index("s")
    w = c * _NS + s
    pltpu.sync_copy(idx_hbm.at[w], idx_v)
    pltpu.sync_copy(oidx_hbm.at[s], oidx_v)
    _sc_table(c2, idx_v, oidx_v, rows_v, acc_v, 0, s * _M, gsem, ssem, asem)
    pltpu.sync_copy(acc_v.at[pl.ds(s * _M, _M)], m_out.at[pl.ds(w * _M, _M)])


_SC_MESH = dict(core_axis_name="c", subcore_axis_name="s")


def _sc_gather1(idx, oidx, dec, C0, C1):
    fn = pl.kernel(
        _sc1_body,
        mesh=plsc.VectorSubcoreMesh(**_SC_MESH),
        out_type=[jax.ShapeDtypeStruct((2, _B * _M, _D), jnp.float32),
                  jax.ShapeDtypeStruct((_B, _D), jnp.float32)],
        scratch_types=[pltpu.VMEM((2 * _TOK, _CHUNK), jnp.int32),
                       pltpu.VMEM((2, _NCHUNK, _CHUNK), jnp.int32),
                       pltpu.VMEM((1, _B), jnp.int32),
                       pltpu.VMEM((4, _CHUNK, _D), jnp.float32),
                       pltpu.VMEM_SHARED((2 * _ROWS_SC, _D), jnp.float32),
                       pltpu.SemaphoreType.DMA,
                       pltpu.SemaphoreType.DMA,
                       pltpu.SemaphoreType.DMA,
                       pltpu.SemaphoreType.DMA],
    )
    return fn(idx, oidx, dec, C0, C1)


def _sc_gather2(idx, oidx, C2):
    fn = pl.kernel(
        _sc2_body,
        mesh=plsc.VectorSubcoreMesh(**_SC_MESH),
        out_type=jax.ShapeDtypeStruct((_B * _M, _D), jnp.float32),
        scratch_types=[pltpu.VMEM((2 * _TOK, _CHUNK), jnp.int32),
                       pltpu.VMEM((2, _NCHUNK, _CHUNK), jnp.int32),
                       pltpu.VMEM((4, _CHUNK, _D), jnp.float32),
                       pltpu.VMEM_SHARED((_ROWS_SC, _D), jnp.float32),
                       pltpu.SemaphoreType.DMA,
                       pltpu.SemaphoreType.DMA,
                       pltpu.SemaphoreType.DMA,
                       pltpu.SemaphoreType.DMA],
    )
    return fn(idx, oidx, C2)


def _dense1_body(x_ref, h0_ref, roots_ref, bias_ref, m_ref,
                 wq_ref, wk_ref, wv_ref, wih_ref, whh_ref, bih_ref, bhh_ref,
                 cur_ref, uo_ref):
    h0 = h0_ref[...]
    q = lax.dot(h0, wq_ref[...], preferred_element_type=jnp.float32)
    roots = roots_ref[...]
    roots2 = roots.reshape(_B * _NT, _D)
    rk = lax.dot(roots2, wk_ref[...], preferred_element_type=jnp.float32)
    rv = lax.dot(roots2, wv_ref[...], preferred_element_type=jnp.float32)
    rk = rk.reshape(_B, _NT, _D)
    rv = rv.reshape(_B, _NT, _D)
    rk_b = rk.astype(jnp.bfloat16).astype(jnp.float32)
    q_b = q.astype(jnp.bfloat16).astype(jnp.float32)
    al = jnp.sum(rk_b * q_b[:, None, :], axis=2) + bias_ref[...]
    aw = jax.nn.softmax(al, axis=1)
    kb = jnp.sum(aw[:, :, None] * rv, axis=1)

    x = x_ref[...]
    gi = lax.dot(x, wih_ref[...], preferred_element_type=jnp.float32) + bih_ref[...]
    gh = lax.dot(h0, whh_ref[...], preferred_element_type=jnp.float32) + bhh_ref[...]
    r = jax.nn.sigmoid(gi[:, 0:_D] + gh[:, 0:_D])
    z = jax.nn.sigmoid(gi[:, _D:2 * _D] + gh[:, _D:2 * _D])
    n = jnp.tanh(gi[:, 2 * _D:3 * _D] + r * gh[:, 2 * _D:3 * _D])
    hidden = (1.0 - z) * n + z * h0
    u = hidden + kb
    cur_ref[...] = u

    logits = jnp.sum(m_ref[0] * u[:, None, :], axis=2)
    prob = jax.nn.softmax(logits, axis=1)
    o_k = jnp.sum(m_ref[1] * prob[:, :, None], axis=1)
    uo_ref[:, 0:_D] = u
    uo_ref[:, _D:2 * _D] = o_k


def _dense1(x, h0, roots, bias, m01, Wq, Wk, Wv, W_ih, W_hh, b_ih2, b_hh2):
    return pl.pallas_call(
        _dense1_body,
        out_shape=[jax.ShapeDtypeStruct((_B, _D), jnp.float32),
                   jax.ShapeDtypeStruct((_B, 2 * _D), jnp.float32)],
    )(x, h0, roots, bias, m01, Wq, Wk, Wv, W_ih, W_hh, b_ih2, b_hh2)


def _dense2_body(uo_ref, m01_ref, m2_ref, pptr_ref):
    u1 = uo_ref[:, 0:_D] + uo_ref[:, _D:2 * _D]
    m1 = m01_ref[1]
    logits1 = jnp.sum(m1 * u1[:, None, :], axis=2)
    prob1 = jax.nn.softmax(logits1, axis=1)
    m2 = m2_ref[...]
    o_k1 = jnp.sum(m2 * prob1[:, :, None], axis=1)
    u2 = u1 + o_k1
    pptr_ref[...] = jnp.sum(m2 * u2[:, None, :], axis=2)


def _dense2(uo, m01, m2):
    return pl.pallas_call(
        _dense2_body,
        out_shape=jax.ShapeDtypeStruct((_B, _M), jnp.float32),
    )(uo, m01, m2.reshape(_B, _M, _D))


def _vocab_body(uo_ref, w_ref, b_ref, out_ref, e_ref, mt_ref, m_ref, s_ref):
    i = pl.program_id(0)

    @pl.when(i == 0)
    def _():
        m_ref[...] = jnp.full((_B, 128), -3e38, jnp.float32)
        s_ref[...] = jnp.zeros((_B, 128), jnp.float32)

    @pl.when(i < _NV)
    def _():
        logits = lax.dot_general(uo_ref[...], w_ref[...],
                                 (((1,), (1,)), ((), ())),
                                 preferred_element_type=jnp.float32)
        logits = logits + b_ref[...]
        t_max = jnp.max(logits, axis=1, keepdims=True)
        m_old = m_ref[...]
        m_new = jnp.maximum(m_old, t_max)
        e = jnp.exp(logits - m_new[:, :1])
        e_ref[i] = e
        mt_ref[i] = m_new
        ssum = jnp.sum(e, axis=1, keepdims=True)
        s_ref[...] = s_ref[...] * jnp.exp(m_old - m_new) + ssum
        m_ref[...] = m_new

    @pl.when(i >= _NV)
    def _():
        j = i - _NV
        scale = jnp.exp(mt_ref[j, :, 0:1] - m_ref[:, 0:1]) / s_ref[:, 0:1]
        out_ref[...] = e_ref[j] * scale


def _vocab(uo, W1t, b12):
    return pl.pallas_call(
        _vocab_body,
        grid=(2 * _NV,),
        in_specs=[
            pl.BlockSpec((_B, 2 * _D), lambda i: (0, 0)),
            pl.BlockSpec((_TV, 2 * _D), lambda i: (lax.min(i, _NV - 1), 0)),
            pl.BlockSpec((1, _TV), lambda i: (0, lax.min(i, _NV - 1))),
        ],
        out_specs=pl.BlockSpec((_B, _TV), lambda i: (0, lax.max(i - _NV, 0))),
        out_shape=jax.ShapeDtypeStruct((_B, _VOCAB), jnp.float32),
        scratch_shapes=[pltpu.VMEM((_NV, _B, _TV), jnp.float32),
                        pltpu.VMEM((_NV, _B, 128), jnp.float32),
                        pltpu.VMEM((_B, 128), jnp.float32),
                        pltpu.VMEM((_B, 128), jnp.float32)],
    )(uo, W1t, b12)


def kernel(decoder_input, story, hidden_states, roots_embed, attention_bias,
           global_index, C0, C1, C2, C3, Wq, Wk, Wv, W1, b1,
           W_ih, W_hh, b_ih, b_hh):
    story = story.astype(jnp.int32)
    dec = decoder_input.astype(jnp.int32).reshape(1, _B)
    idx = story.transpose(0, 2, 1).reshape(_NW, _TOK * _NCHUNK, _CHUNK)
    oidx = (jnp.arange(2, dtype=jnp.int32)[None, :, None, None] * _ROWS_SC
            + jnp.arange(_NS, dtype=jnp.int32)[:, None, None, None] * _M
            + jnp.arange(_NCHUNK, dtype=jnp.int32)[None, None, :, None] * _CHUNK
            + jnp.arange(_CHUNK, dtype=jnp.int32)[None, None, None, :])
    m01_flat, x = _sc_gather1(idx, oidx, dec, C0, C1)
    m2_flat = _sc_gather2(idx, oidx, C2)
    m01 = m01_flat.reshape(2, _B, _M, _D)
    h0 = hidden_states[0]
    bias = attention_bias[:, :, 0]
    cur, uo = _dense1(x, h0, roots_embed, bias, m01, Wq, Wk, Wv,
                      W_ih, W_hh, b_ih.reshape(1, -1), b_hh.reshape(1, -1))
    b_pad = jnp.pad(b1.reshape(1, -1), ((0, 0), (0, _NV * _TV - _VOCAB)),
                    constant_values=-1e30)
    pvocab = _vocab(uo, W1.T, b_pad)
    uo2, _pv = lax.optimization_barrier((uo, pvocab))
    pptr = _dense2(uo2, m01, m2_flat)
    pvocab = _pv
    return (pptr, pvocab, cur[None])
